# Initial kernel scaffold; baseline (speedup 1.0000x reference)
import jax
import jax.numpy as jnp
from jax.experimental import pallas as pl


def kernel(prefix, embedding_table):
    raise NotImplementedError("write your pallas kernel here")


# speedup vs baseline: 1.8821x; 1.8821x over previous
import functools

import jax
import jax.numpy as jnp
from jax import lax
from jax.experimental import pallas as pl
from jax.experimental.pallas import tpu as pltpu
from jax.experimental.pallas import tpu_sc as plsc

PRE_SEQ_LEN = 64
EMBED_DIM = 49152
BATCH = 32
NUM_ROWS = BATCH * PRE_SEQ_LEN

NUM_CORES = 2
NUM_SUBCORES = 16
NUM_WORKERS = NUM_CORES * NUM_SUBCORES
ROWS_PER_WORKER = NUM_ROWS // NUM_WORKERS

CHUNK_W = 1536
NUM_CHUNKS = EMBED_DIM // CHUNK_W


def _sc_body(pref_hbm, tbl_hbm, out_hbm, idx_v, buf, sem):
    wid = lax.axis_index("s") * NUM_CORES + lax.axis_index("c")
    base = wid * ROWS_PER_WORKER
    pltpu.sync_copy(pref_hbm.at[pl.ds(base, ROWS_PER_WORKER)], idx_v)

    def chunk(c, carry):
        off = c * CHUNK_W
        pltpu.async_copy(tbl_hbm.at[idx_v, pl.ds(off, CHUNK_W)], buf, sem).wait()
        pltpu.sync_copy(
            buf, out_hbm.at[pl.ds(base, ROWS_PER_WORKER), pl.ds(off, CHUNK_W)]
        )
        return carry

    lax.fori_loop(0, NUM_CHUNKS, chunk, 0)


@functools.partial(
    pl.kernel,
    out_type=jax.ShapeDtypeStruct((NUM_ROWS, EMBED_DIM), jnp.float32),
    mesh=plsc.VectorSubcoreMesh(core_axis_name="c", subcore_axis_name="s"),
    scratch_types=[
        pltpu.VMEM((ROWS_PER_WORKER,), jnp.int32),
        pltpu.VMEM((ROWS_PER_WORKER, CHUNK_W), jnp.float32),
        pltpu.SemaphoreType.DMA,
    ],
)
def _gather_rows(pref_hbm, tbl_hbm, out_hbm, idx_v, buf, sem):
    _sc_body(pref_hbm, tbl_hbm, out_hbm, idx_v, buf, sem)


def kernel(prefix, embedding_table):
    flat_idx = prefix.reshape(NUM_ROWS).astype(jnp.int32)
    out = _gather_rows(flat_idx, embedding_table)
    return out.reshape(BATCH, PRE_SEQ_LEN, EMBED_DIM)


# speedup vs baseline: 1.9698x; 1.0466x over previous
import functools

import jax
import jax.numpy as jnp
from jax import lax
from jax.experimental import pallas as pl
from jax.experimental.pallas import tpu as pltpu
from jax.experimental.pallas import tpu_sc as plsc

PRE_SEQ_LEN = 64
EMBED_DIM = 49152
BATCH = 32
NUM_ROWS = BATCH * PRE_SEQ_LEN

NUM_CORES = 2
NUM_SUBCORES = 16
NUM_WORKERS = NUM_CORES * NUM_SUBCORES
ROWS_PER_WORKER = NUM_ROWS // NUM_WORKERS

CHUNK_W = 768
NUM_CHUNKS = EMBED_DIM // CHUNK_W


def _sc_body(pref_hbm, tbl_hbm, out_hbm, idx_v, bufs, gsem):
    wid = lax.axis_index("s") * NUM_CORES + lax.axis_index("c")
    base = wid * ROWS_PER_WORKER
    pltpu.sync_copy(pref_hbm.at[pl.ds(base, ROWS_PER_WORKER)], idx_v)

    def gather(c, slot):
        off = lax.min(c, NUM_CHUNKS - 1) * CHUNK_W
        return pltpu.make_async_copy(
            tbl_hbm.at[idx_v, pl.ds(off, CHUNK_W)], bufs.at[slot], gsem.at[slot]
        )

    gather(0, 0).start()

    def chunk(c, carry):
        slot = lax.rem(c, 2)
        gather(c + 1, 1 - slot).start()
        gather(c, slot).wait()
        pltpu.sync_copy(
            bufs.at[slot],
            out_hbm.at[pl.ds(base, ROWS_PER_WORKER), pl.ds(c * CHUNK_W, CHUNK_W)],
        )
        return carry

    lax.fori_loop(0, NUM_CHUNKS, chunk, 0)
    gather(NUM_CHUNKS - 1, lax.rem(NUM_CHUNKS, 2)).wait()


@functools.partial(
    pl.kernel,
    out_type=jax.ShapeDtypeStruct((NUM_ROWS, EMBED_DIM), jnp.float32),
    mesh=plsc.VectorSubcoreMesh(core_axis_name="c", subcore_axis_name="s"),
    scratch_types=[
        pltpu.VMEM((ROWS_PER_WORKER,), jnp.int32),
        pltpu.VMEM((2, ROWS_PER_WORKER, CHUNK_W), jnp.float32),
        pltpu.SemaphoreType.DMA((2,)),
    ],
)
def _gather_rows(pref_hbm, tbl_hbm, out_hbm, idx_v, bufs, gsem):
    _sc_body(pref_hbm, tbl_hbm, out_hbm, idx_v, bufs, gsem)


def kernel(prefix, embedding_table):
    flat_idx = prefix.reshape(NUM_ROWS).astype(jnp.int32)
    out = _gather_rows(flat_idx, embedding_table)
    return out.reshape(BATCH, PRE_SEQ_LEN, EMBED_DIM)


# speedup vs baseline: 2.9827x; 1.5142x over previous
import functools

import jax
import jax.numpy as jnp
from jax import lax
from jax.experimental import pallas as pl
from jax.experimental.pallas import tpu as pltpu
from jax.experimental.pallas import tpu_sc as plsc

PRE_SEQ_LEN = 64
EMBED_DIM = 49152
BATCH = 32
NUM_ROWS = BATCH * PRE_SEQ_LEN
TBL_ROWS = PRE_SEQ_LEN

NUM_CORES = 2
NUM_SUBCORES = 16
LANES = 16

NUM_PHASES = 4
PHASE_W = EMBED_DIM // (NUM_CORES * NUM_PHASES)
STAGE_ROWS = TBL_ROWS // NUM_SUBCORES

ROWS_PER_WORKER = NUM_ROWS // NUM_SUBCORES
NUM_GROUPS = ROWS_PER_WORKER // LANES


def _sc_body(pref_hbm, tbl_hbm, out_hbm, idx_v, spmem, sem):
    core = lax.axis_index("c")
    sub = lax.axis_index("s")
    base = sub * ROWS_PER_WORKER
    pltpu.sync_copy(pref_hbm.at[pl.ds(base, ROWS_PER_WORKER)], idx_v)
    srow = sub * STAGE_ROWS

    for phase in range(NUM_PHASES):
        qbase = (core * NUM_PHASES + phase) * PHASE_W

        pltpu.sync_copy(
            tbl_hbm.at[pl.ds(srow, STAGE_ROWS), pl.ds(qbase, PHASE_W)],
            spmem.at[pl.ds(srow, STAGE_ROWS)],
        )
        plsc.subcore_barrier()

        def fire(g):
            ivec = idx_v[pl.ds(g * LANES, LANES)]
            for lane in range(LANES):
                s = ivec[lane]
                row = base + g * LANES + lane
                pltpu.make_async_copy(
                    spmem.at[pl.ds(s, 1), pl.ds(0, PHASE_W)],
                    out_hbm.at[pl.ds(row, 1), pl.ds(qbase, PHASE_W)],
                    sem,
                ).start()

        def drain_batch():
            for _ in range(LANES):
                pltpu.make_async_copy(
                    spmem.at[pl.ds(0, 1), pl.ds(0, PHASE_W)],
                    out_hbm.at[pl.ds(base, 1), pl.ds(qbase, PHASE_W)],
                    sem,
                ).wait()

        fire(0)

        def step(g, carry):
            fire(g)
            drain_batch()
            return carry

        lax.fori_loop(1, NUM_GROUPS, step, 0)
        drain_batch()
        plsc.subcore_barrier()


@functools.partial(
    pl.kernel,
    out_type=jax.ShapeDtypeStruct((NUM_ROWS, EMBED_DIM), jnp.float32),
    mesh=plsc.VectorSubcoreMesh(core_axis_name="c", subcore_axis_name="s"),
    scratch_types=[
        pltpu.VMEM((ROWS_PER_WORKER,), jnp.int32),
        pltpu.VMEM_SHARED((TBL_ROWS, PHASE_W), jnp.float32),
        pltpu.SemaphoreType.DMA,
    ],
)
def _gather_rows(pref_hbm, tbl_hbm, out_hbm, idx_v, spmem, sem):
    _sc_body(pref_hbm, tbl_hbm, out_hbm, idx_v, spmem, sem)


def kernel(prefix, embedding_table):
    flat_idx = prefix.reshape(NUM_ROWS).astype(jnp.int32)
    out = _gather_rows(flat_idx, embedding_table)
    return out.reshape(BATCH, PRE_SEQ_LEN, EMBED_DIM)


# speedup vs baseline: 3.0000x; 1.0058x over previous
import functools

import jax
import jax.numpy as jnp
from jax import lax
from jax.experimental import pallas as pl
from jax.experimental.pallas import tpu as pltpu
from jax.experimental.pallas import tpu_sc as plsc

PRE_SEQ_LEN = 64
EMBED_DIM = 49152
BATCH = 32
NUM_ROWS = BATCH * PRE_SEQ_LEN
TBL_ROWS = PRE_SEQ_LEN

NUM_CORES = 2
NUM_SUBCORES = 16
LANES = 16

NUM_PHASES = 3
PHASE_W = EMBED_DIM // (NUM_CORES * NUM_PHASES)
STAGE_ROWS = TBL_ROWS // NUM_SUBCORES

ROWS_PER_WORKER = NUM_ROWS // NUM_SUBCORES
NUM_GROUPS = ROWS_PER_WORKER // LANES


def _sc_body(pref_hbm, tbl_hbm, out_hbm, idx_v, spmem, sem):
    core = lax.axis_index("c")
    sub = lax.axis_index("s")
    base = sub * ROWS_PER_WORKER
    pltpu.sync_copy(pref_hbm.at[pl.ds(base, ROWS_PER_WORKER)], idx_v)
    srow = sub * STAGE_ROWS

    for phase in range(NUM_PHASES):
        qbase = (core * NUM_PHASES + phase) * PHASE_W

        pltpu.sync_copy(
            tbl_hbm.at[pl.ds(srow, STAGE_ROWS), pl.ds(qbase, PHASE_W)],
            spmem.at[pl.ds(srow, STAGE_ROWS)],
        )
        plsc.subcore_barrier()

        def fire(g):
            ivec = idx_v[pl.ds(g * LANES, LANES)]
            for lane in range(LANES):
                s = ivec[lane]
                row = base + g * LANES + lane
                pltpu.make_async_copy(
                    spmem.at[pl.ds(s, 1), pl.ds(0, PHASE_W)],
                    out_hbm.at[pl.ds(row, 1), pl.ds(qbase, PHASE_W)],
                    sem,
                ).start()

        def drain_batch():
            for _ in range(LANES):
                pltpu.make_async_copy(
                    spmem.at[pl.ds(0, 1), pl.ds(0, PHASE_W)],
                    out_hbm.at[pl.ds(base, 1), pl.ds(qbase, PHASE_W)],
                    sem,
                ).wait()

        fire(0)

        def step(g, carry):
            fire(g)
            drain_batch()
            return carry

        lax.fori_loop(1, NUM_GROUPS, step, 0)
        drain_batch()
        plsc.subcore_barrier()


@functools.partial(
    pl.kernel,
    out_type=jax.ShapeDtypeStruct((NUM_ROWS, EMBED_DIM), jnp.float32),
    mesh=plsc.VectorSubcoreMesh(core_axis_name="c", subcore_axis_name="s"),
    scratch_types=[
        pltpu.VMEM((ROWS_PER_WORKER,), jnp.int32),
        pltpu.VMEM_SHARED((TBL_ROWS, PHASE_W), jnp.float32),
        pltpu.SemaphoreType.DMA,
    ],
)
def _gather_rows(pref_hbm, tbl_hbm, out_hbm, idx_v, spmem, sem):
    _sc_body(pref_hbm, tbl_hbm, out_hbm, idx_v, spmem, sem)


def kernel(prefix, embedding_table):
    flat_idx = prefix.reshape(NUM_ROWS).astype(jnp.int32)
    out = _gather_rows(flat_idx, embedding_table)
    return out.reshape(BATCH, PRE_SEQ_LEN, EMBED_DIM)
